# Initial kernel scaffold; baseline (speedup 1.0000x reference)
import jax
import jax.numpy as jnp
from jax.experimental import pallas as pl


def kernel(x, edge_index, edge_attr, node_embed_W, node_embed_b, edge_embed_W, edge_embed_b, lin_node_W, lin_node_b, lin_edge_W, lin_edge_b, adm_W, adm_b, ln_g, ln_b):
    raise NotImplementedError("write your pallas kernel here")


# speedup vs baseline: 2.1989x; 2.1989x over previous
import functools

import jax
import jax.numpy as jnp
import numpy as np
from jax import lax
from jax.experimental import pallas as pl
from jax.experimental.pallas import tpu as pltpu
from jax.experimental.pallas import tpu_sc as plsc

N = 10000
E = 320000
HID = 128
HHID = HID // 2
EAW = 16
GW = 2 * EAW
LAYERS = 4
NC = 2
NS = 16
EPT = E // NS
CH = 80
TCH = EPT // CH
NP = 10112
RPT = NP // NS

_MESH = plsc.VectorSubcoreMesh(
    core_axis_name="c", subcore_axis_name="s", num_cores=NC, num_subcores=NS)

_SC_PARAMS = pltpu.CompilerParams(
    use_tc_tiling_on_sc=False, needs_layout_passes=False)

_GDN = lax.GatherDimensionNumbers(
    offset_dims=(), collapsed_slice_dims=(0,), start_index_map=(0,))


def _bcast_lane(vec, k):
  idx = jnp.full((16, 1), k, jnp.int32)
  return lax.gather(vec, idx, _GDN, (1,),
                    mode=lax.GatherScatterMode.PROMISE_IN_BOUNDS)


def _zero_fill(buf, rows, vregs_per_row):
  z = jnp.zeros((16,), jnp.float32)

  def body(i, carry):
    for c in range(vregs_per_row):
      buf[i, pl.ds(c * 16, 16)] = z
    return carry

  lax.fori_loop(0, rows, body, 0)


def _zero_acc(acc, zbuf, tbase):
  for k in range(4):
    pltpu.sync_copy(zbuf, acc.at[pl.ds(tbase + k * 128, 128)])
  pltpu.sync_copy(zbuf.at[pl.ds(0, 120)], acc.at[pl.ds(tbase + 512, 120)])


def _write_out(acc, out, tbase):
  for k in range(4):
    sl = pl.ds(tbase + k * 128, 128)
    pltpu.sync_copy(acc.at[sl], out.at[sl])
  sl = pl.ds(tbase + 512, 120)
  pltpu.sync_copy(acc.at[sl], out.at[sl])


def _make_spmv_kernel():
  @functools.partial(
      pl.kernel,
      out_type=[
          jax.ShapeDtypeStruct((NC, NP, HHID), jnp.float32),
          jax.ShapeDtypeStruct((NP, GW), jnp.float32),
      ],
      mesh=_MESH,
      compiler_params=_SC_PARAMS,
      name="spmv_sc",
      scratch_types=[
          pltpu.VMEM((128, HHID), jnp.float32),
          pltpu.VMEM((128, GW), jnp.float32),
          pltpu.VMEM((CH, HHID), jnp.float32),
          pltpu.VMEM((CH, EAW), jnp.float32),
          pltpu.VMEM((CH, GW), jnp.float32),
          pltpu.VMEM((TCH, CH), jnp.int32),
          pltpu.VMEM((TCH, CH), jnp.int32),
          pltpu.VMEM((CH,), jnp.float32),
          pltpu.VMEM_SHARED((NP, HHID), jnp.float32),
          pltpu.VMEM_SHARED((NP, GW), jnp.float32),
          pltpu.SemaphoreType.DMA,
      ],
  )
  def spmv_kernel(h_hbm, ea_hbm, src_hbm, dst_hbm, y_hbm, out_hbm, out2_hbm,
                  zbuf, zbuf2, rows, eab, pay, srcb, dstb, yvm, acc,
                  acc2, sem):
    core = lax.axis_index("c")
    sid = lax.axis_index("s")
    tbase = sid * RPT
    hview = h_hbm.at[core]
    on_core0 = core == 0

    _zero_fill(zbuf, 128, HHID // 16)
    _zero_acc(acc, zbuf, tbase)

    @pl.when(on_core0)
    def _():
      _zero_fill(zbuf2, 128, GW // 16)
      _zero_acc(acc2, zbuf2, tbase)

    pltpu.sync_copy(src_hbm.at[sid], srcb)
    pltpu.sync_copy(dst_hbm.at[sid], dstb)
    plsc.subcore_barrier()

    def chunk(j, carry):
      pltpu.async_copy(hview.at[srcb.at[j]], rows, sem).wait()
      pltpu.sync_copy(y_hbm.at[sid, pl.ds(j * CH, CH)], yvm)

      for g in range(CH // 16):
        yg = yvm[pl.ds(g * 16, 16)]
        for k in range(16):
          i = g * 16 + k
          yv = _bcast_lane(yg, k)
          for c in range(HHID // 16):
            sl = pl.ds(c * 16, 16)
            rows[i, sl] = rows[i, sl] * yv

      pltpu.sync_copy(rows, acc.at[dstb.at[j]], add=True)

      @pl.when(on_core0)
      def _():
        pltpu.sync_copy(ea_hbm.at[sid, j], eab)
        for g in range(CH // 16):
          yg = yvm[pl.ds(g * 16, 16)]
          for k in range(16):
            i = g * 16 + k
            yv = _bcast_lane(yg, k)
            pay[i, pl.ds(0, EAW)] = eab[i, :] * yv
            pay[i, pl.ds(EAW, EAW)] = yv
        pltpu.sync_copy(pay, acc2.at[dstb.at[j]], add=True)

      return carry

    lax.fori_loop(0, TCH, chunk, 0)
    plsc.subcore_barrier()
    _write_out(acc, out_hbm.at[core], tbase)

    @pl.when(on_core0)
    def _():
      _write_out(acc2, out2_hbm, tbase)

  return spmv_kernel


_spmv_call = _make_spmv_kernel()


_BN = 2000
_BE = 6400


def _embed_body(x_ref, w_ref, b_ref, o_ref, os_ref):
  h = jnp.dot(x_ref[...], w_ref[...],
              preferred_element_type=jnp.float32) + b_ref[...]
  o_ref[...] = h
  os_ref[0] = h[:, :HHID]
  os_ref[1] = h[:, HHID:]


def _node_embed(x, w, b):
  return pl.pallas_call(
      _embed_body,
      name="node_embed_tc",
      grid=(N // _BN,),
      in_specs=[
          pl.BlockSpec((_BN, HID), lambda i: (i, 0)),
          pl.BlockSpec((HID, HID), lambda i: (0, 0)),
          pl.BlockSpec((1, HID), lambda i: (0, 0)),
      ],
      out_specs=[
          pl.BlockSpec((_BN, HID), lambda i: (i, 0)),
          pl.BlockSpec((NC, _BN, HHID), lambda i: (0, i, 0)),
      ],
      out_shape=[
          jax.ShapeDtypeStruct((N, HID), jnp.float32),
          jax.ShapeDtypeStruct((NC, N, HHID), jnp.float32),
      ],
  )(x, w, b.reshape(1, HID))


def _edge_body(ea_ref, c_ref, d_ref, yt_ref):
  y = jax.nn.sigmoid(
      jnp.dot(ea_ref[...], c_ref[...], preferred_element_type=jnp.float32)
      + d_ref[...])
  yt_ref[...] = y.T


def _edge_pre(ea, cmat, dvec):
  return pl.pallas_call(
      _edge_body,
      name="edge_pre_tc",
      grid=(E // _BE,),
      in_specs=[
          pl.BlockSpec((_BE, EAW), lambda i: (i, 0)),
          pl.BlockSpec((EAW, LAYERS), lambda i: (0, 0)),
          pl.BlockSpec((1, LAYERS), lambda i: (0, 0)),
      ],
      out_specs=pl.BlockSpec((LAYERS, _BE), lambda i: (0, i)),
      out_shape=jax.ShapeDtypeStruct((LAYERS, E), jnp.float32),
  )(ea, cmat, dvec.reshape(1, LAYERS))


def _layer_body(h_ref, p_ref, g2_ref, wn_ref, wc_ref, bc_ref, g_ref, b_ref,
                o_ref, os_ref):
  p = jnp.concatenate([p_ref[0], p_ref[1]], axis=1)
  ga = g2_ref[:, 0:EAW]
  gy = g2_ref[:, EAW][:, None]
  agg = (jnp.dot(p, wn_ref[...], preferred_element_type=jnp.float32)
         + jnp.dot(ga, wc_ref[...], preferred_element_type=jnp.float32)
         + gy * bc_ref[...])
  mu = jnp.mean(agg, axis=1, keepdims=True)
  var = jnp.mean((agg - mu) ** 2, axis=1, keepdims=True)
  xn = (agg - mu) * lax.rsqrt(var + 1e-5) * g_ref[...] + b_ref[...]
  h = h_ref[...] + jnp.maximum(xn, 0.0)
  o_ref[...] = h
  os_ref[0] = h[:, :HHID]
  os_ref[1] = h[:, HHID:]


def _layer_update(h, p2, ga2, wn, wc, bc, g, b):
  return pl.pallas_call(
      _layer_body,
      name="layer_tc",
      grid=(N // _BN,),
      in_specs=[
          pl.BlockSpec((_BN, HID), lambda i: (i, 0)),
          pl.BlockSpec((NC, _BN, HHID), lambda i: (0, i, 0)),
          pl.BlockSpec((_BN, GW), lambda i: (i, 0)),
          pl.BlockSpec((HID, HID), lambda i: (0, 0)),
          pl.BlockSpec((EAW, HID), lambda i: (0, 0)),
          pl.BlockSpec((1, HID), lambda i: (0, 0)),
          pl.BlockSpec((1, HID), lambda i: (0, 0)),
          pl.BlockSpec((1, HID), lambda i: (0, 0)),
      ],
      out_specs=[
          pl.BlockSpec((_BN, HID), lambda i: (i, 0)),
          pl.BlockSpec((NC, _BN, HHID), lambda i: (0, i, 0)),
      ],
      out_shape=[
          jax.ShapeDtypeStruct((N, HID), jnp.float32),
          jax.ShapeDtypeStruct((NC, N, HHID), jnp.float32),
      ],
  )(h, p2, ga2, wn, wc, bc.reshape(1, HID), g.reshape(1, HID),
    b.reshape(1, HID))


def kernel(x, edge_index, edge_attr, node_embed_W, node_embed_b,
           edge_embed_W, edge_embed_b, lin_node_W, lin_node_b,
           lin_edge_W, lin_edge_b, adm_W, adm_b, ln_g, ln_b):
  src3d = edge_index[0].reshape(NS, TCH, CH)
  dst3d = edge_index[1].reshape(NS, TCH, CH)
  ea3d = edge_attr.reshape(NS, TCH, CH, EAW)

  a = adm_W[:, :, 0].T
  cmat = edge_embed_W @ a
  dvec = edge_embed_b @ a + adm_b[:, 0]
  wc = jnp.einsum("ij,ljk->lik", edge_embed_W, lin_edge_W)
  bc = (jnp.einsum("j,ljk->lk", edge_embed_b, lin_edge_W)
        + lin_node_b + lin_edge_b)

  h, hsplit = _node_embed(x, node_embed_W, node_embed_b)
  yt = _edge_pre(edge_attr, cmat, dvec)

  for l in range(LAYERS):
    p2, ga2 = _spmv_call(hsplit, ea3d, src3d, dst3d,
                         yt[l].reshape(NS, EPT))
    h, hsplit = _layer_update(h, p2, ga2, lin_node_W[l], wc[l], bc[l],
                              ln_g[l], ln_b[l])
  return h
